# baseline (speedup 1.0000x reference)
import functools

import jax
import jax.numpy as jnp
from jax import lax
from jax.experimental import pallas as pl
from jax.experimental.pallas import tpu as pltpu
from jax.experimental.pallas import tpu_sc as plsc

N = 10000
E = 320000
IN_CH = 128
OUT_CH = 32
HEADS = 2
EAD = 4
C = HEADS * EAD
F = OUT_CH
NEG = 0.2

CH = 128
NCHUNK = E // CH
NSC = 2
NT = 16
NW = NSC * NT

_ROW_BLK = 1000

_mesh = plsc.VectorSubcoreMesh(core_axis_name="c", subcore_axis_name="s")
_sc_params = pltpu.CompilerParams(needs_layout_passes=False,
                                  use_tc_tiling_on_sc=False)


def _tc_body(x_ref, w_ref, w12_ref, xws_ref, s1_ref, s2_ref):
    xw = jnp.dot(x_ref[...], w_ref[...], preferred_element_type=jnp.float32)
    s = jnp.dot(xw, w12_ref[...], preferred_element_type=jnp.float32)
    xws_ref[0] = xw[:, :128]
    xws_ref[1] = xw[:, 128:]
    s1_ref[...] = s[:, :16]
    s2_ref[...] = s[:, 16:]


def _dense_stage(x, weight, w12p):
    grid = N // _ROW_BLK
    return pl.pallas_call(
        _tc_body,
        grid=(grid,),
        in_specs=[
            pl.BlockSpec((_ROW_BLK, IN_CH), lambda i: (i, 0)),
            pl.BlockSpec((IN_CH, C * F), lambda i: (0, 0)),
            pl.BlockSpec((C * F, 32), lambda i: (0, 0)),
        ],
        out_specs=[
            pl.BlockSpec((2, _ROW_BLK, 128), lambda i: (0, i, 0)),
            pl.BlockSpec((_ROW_BLK, 16), lambda i: (i, 0)),
            pl.BlockSpec((_ROW_BLK, 16), lambda i: (i, 0)),
        ],
        out_shape=[
            jax.ShapeDtypeStruct((2, N, 128), jnp.float32),
            jax.ShapeDtypeStruct((N, 16), jnp.float32),
            jax.ShapeDtypeStruct((N, 16), jnp.float32),
        ],
    )(x, weight, w12p)


_SLC_A = 624
_LAST_A = N - 15 * _SLC_A


def _sca_body(row_r, col_r, ea_r, s1_r, s2_r, z8_r,
              ex_r, ssp_r,
              ssum_sp, rowb, colb, eab, g1, g2, exb):
    c = lax.axis_index("c")
    s = lax.axis_index("s")
    wid = s * NSC + c
    iota = lax.iota(jnp.int32, 16)
    lane8 = iota & 7
    ea_lane = (iota >> 1) & 3
    low_mask = iota < 8

    @pl.when(s < 15)
    def _():
        pltpu.sync_copy(z8_r.at[pl.ds(s * _SLC_A, _SLC_A)],
                        ssum_sp.at[pl.ds(s * _SLC_A, _SLC_A)])

    @pl.when(s == 15)
    def _():
        pltpu.sync_copy(z8_r.at[pl.ds(15 * _SLC_A, _LAST_A)],
                        ssum_sp.at[pl.ds(15 * _SLC_A, _LAST_A)])

    plsc.subcore_barrier()

    def chunk_body(j, _):
        cid = wid + NW * j
        e0 = cid * CH
        pltpu.sync_copy(row_r.at[pl.ds(e0, CH)], rowb)
        pltpu.sync_copy(col_r.at[pl.ds(e0, CH)], colb)
        pltpu.sync_copy(ea_r.at[pl.ds(e0, CH)], eab)
        pltpu.sync_copy(s1_r.at[rowb], g1)
        pltpu.sync_copy(s2_r.at[colb], g2)

        def edge(e, _):
            ev = jnp.broadcast_to(e, (16,))
            t = (plsc.load_gather(g1, [ev, iota])
                 + plsc.load_gather(g2, [ev, iota]))
            sc = plsc.load_gather(eab, [ev, ea_lane])
            a = t * sc
            a = jnp.maximum(a, NEG * a)
            ex = jnp.exp(a)
            plsc.store_scatter(exb, [ev, lane8], ex, mask=low_mask)
            return 0

        lax.fori_loop(0, CH, edge, 0, unroll=4)
        pltpu.sync_copy(exb, ex_r.at[pl.ds(e0, CH)])
        pltpu.sync_copy(exb, ssum_sp.at[rowb], add=True)
        return 0

    nj = jnp.where(wid < NCHUNK - (NCHUNK // NW) * NW,
                   NCHUNK // NW + 1, NCHUNK // NW)
    lax.fori_loop(0, nj, chunk_body, 0)
    plsc.subcore_barrier()

    def flush(r0, nrow):
        pltpu.sync_copy(ssum_sp.at[pl.ds(r0, nrow)],
                        ssp_r.at[c, pl.ds(r0, nrow)])

    @pl.when(s < 15)
    def _():
        flush(s * _SLC_A, _SLC_A)

    @pl.when(s == 15)
    def _():
        flush(15 * _SLC_A, _LAST_A)


_sc_a = functools.partial(
    pl.kernel,
    out_type=[
        jax.ShapeDtypeStruct((E, C), jnp.float32),
        jax.ShapeDtypeStruct((2, N, C), jnp.float32),
    ],
    mesh=_mesh,
    compiler_params=_sc_params,
    scratch_types=[
        pltpu.VMEM_SHARED((N, C), jnp.float32),
        pltpu.VMEM((CH,), jnp.int32),
        pltpu.VMEM((CH,), jnp.int32),
        pltpu.VMEM((CH, 4), jnp.float32),
        pltpu.VMEM((CH, 16), jnp.float32),
        pltpu.VMEM((CH, 16), jnp.float32),
        pltpu.VMEM((CH, 8), jnp.float32),
    ],
)(_sca_body)


_EPI = 125


def _scc_body(row_r, col_r, ex_r, ssp_r, xws_r, bias_r, z128_r,
              o_r, ap_r,
              out_sp, rowb, colb, exb, d0buf, d1buf, gbuf, apb, biasb):
    c = lax.axis_index("c")
    s = lax.axis_index("s")
    iota = lax.iota(jnp.int32, 16)
    lane8 = iota & 7
    lane_e = iota >> 3
    c4 = c * 4
    amask = (lane8 >= c4) & (lane8 < c4 + 4)

    pltpu.sync_copy(z128_r.at[pl.ds(s * 625, 625)],
                    out_sp.at[pl.ds(s * 625, 625)])
    pltpu.sync_copy(bias_r.at[pl.ds(c * 128, 128)], biasb)
    plsc.subcore_barrier()

    def chunk_body(j, _):
        cid = s + NT * j
        e0 = cid * CH
        pltpu.sync_copy(row_r.at[pl.ds(e0, CH)], rowb)
        pltpu.sync_copy(col_r.at[pl.ds(e0, CH)], colb)
        pltpu.sync_copy(ex_r.at[pl.ds(e0, CH)], exb)
        pltpu.sync_copy(ssp_r.at[0].at[rowb], d0buf)
        pltpu.sync_copy(ssp_r.at[1].at[rowb], d1buf)

        pltpu.sync_copy(xws_r.at[c].at[colb], gbuf)

        def pair(k, _):
            e = k * 2
            ev = jnp.broadcast_to(e, (16,)) + lane_e
            exv = plsc.load_gather(exb, [ev, lane8])
            d = (plsc.load_gather(d0buf, [ev, lane8])
                 + plsc.load_gather(d1buf, [ev, lane8]) + 1e-16)
            al = exv / d
            plsc.store_scatter(apb, [ev, lane8 - c4], al, mask=amask)
            return 0

        lax.fori_loop(0, CH // 2, pair, 0, unroll=4)

        def scale(e, _):
            erow = jnp.broadcast_to(e, (16,))
            for h in range(4):
                m = plsc.load_gather(
                    apb, [erow, jnp.broadcast_to(h, (16,))])
                for r in (2 * h, 2 * h + 1):
                    colv = r * 16 + iota
                    g = plsc.load_gather(gbuf, [erow, colv])
                    plsc.store_scatter(gbuf, [erow, colv], g * m)
            return 0

        lax.fori_loop(0, CH, scale, 0, unroll=2)

        pltpu.sync_copy(apb, ap_r.at[c, pl.ds(e0, CH)])

        pltpu.sync_copy(gbuf, out_sp.at[rowb], add=True)
        return 0

    nj = jnp.where(s < NCHUNK - (NCHUNK // NT) * NT,
                   NCHUNK // NT + 1, NCHUNK // NT)
    lax.fori_loop(0, nj, chunk_body, 0)
    plsc.subcore_barrier()

    def batch(b, _):
        r0 = s * 625 + b * _EPI
        pltpu.sync_copy(out_sp.at[pl.ds(r0, _EPI)], gbuf.at[pl.ds(0, _EPI)])

        def browe(m, _):
            mrow = jnp.broadcast_to(m, (16,))
            for r in range(8):
                colv = r * 16 + iota
                g = plsc.load_gather(gbuf, [mrow, colv])
                plsc.store_scatter(gbuf, [mrow, colv],
                                   g + biasb[pl.ds(r * 16, 16)])
            return 0

        lax.fori_loop(0, _EPI, browe, 0, unroll=2)

        pltpu.sync_copy(gbuf.at[pl.ds(0, _EPI)], o_r.at[c, pl.ds(r0, _EPI)])
        return 0

    lax.fori_loop(0, 5, batch, 0)


_sc_c = functools.partial(
    pl.kernel,
    out_type=[
        jax.ShapeDtypeStruct((2, N, 128), jnp.float32),
        jax.ShapeDtypeStruct((2, E, 4), jnp.float32),
    ],
    mesh=_mesh,
    compiler_params=_sc_params,
    scratch_types=[
        pltpu.VMEM_SHARED((N, 128), jnp.float32),
        pltpu.VMEM((CH,), jnp.int32),
        pltpu.VMEM((CH,), jnp.int32),
        pltpu.VMEM((CH, 8), jnp.float32),
        pltpu.VMEM((CH, 8), jnp.float32),
        pltpu.VMEM((CH, 8), jnp.float32),
        pltpu.VMEM((CH, 128), jnp.float32),
        pltpu.VMEM((CH, 4), jnp.float32),
        pltpu.VMEM((128,), jnp.float32),
    ],
)(_scc_body)


def kernel(x, edge_index, edge_attr, weight, att_weight, bias):
    aw1 = att_weight[0, :, :F]
    aw2 = att_weight[0, :, F:]
    eye = jnp.eye(C, dtype=jnp.float32)
    w1 = (aw1[:, :, None] * eye[:, None, :]).reshape(C * F, C)
    w2 = (aw2[:, :, None] * eye[:, None, :]).reshape(C * F, C)
    zc = jnp.zeros((C * F, C), jnp.float32)
    w12p = jnp.concatenate([w1, zc, w2, zc], axis=1)

    xws, s1p, s2p = _dense_stage(x, weight, w12p)

    row = edge_index[0]
    col = edge_index[1]
    z8 = jnp.zeros((N, C), jnp.float32)
    z128 = jnp.zeros((N, 128), jnp.float32)

    ex, ssp = _sc_a(row, col, edge_attr, s1p, s2p, z8)
    o2, ap2 = _sc_c(row, col, ex, ssp, xws, bias, z128)

    out = jnp.concatenate([o2[0], o2[1]], axis=1)
    alpha = jnp.concatenate([ap2[0], ap2[1]], axis=1)
    return (out, alpha)


# speedup vs baseline: 1.1544x; 1.1544x over previous
import functools

import jax
import jax.numpy as jnp
from jax import lax
from jax.experimental import pallas as pl
from jax.experimental.pallas import tpu as pltpu
from jax.experimental.pallas import tpu_sc as plsc

N = 10000
E = 320000
IN_CH = 128
OUT_CH = 32
HEADS = 2
EAD = 4
C = HEADS * EAD
F = OUT_CH
NEG = 0.2

CH = 128
NCHUNK = E // CH
NSC = 2
NT = 16
NW = NSC * NT

_ROW_BLK = 1000

_mesh = plsc.VectorSubcoreMesh(core_axis_name="c", subcore_axis_name="s")
_sc_params = pltpu.CompilerParams(needs_layout_passes=False,
                                  use_tc_tiling_on_sc=False)


def _tc_body(x_ref, w_ref, w12_ref, xws_ref, s1_ref, s2_ref):
    xw = jnp.dot(x_ref[...], w_ref[...], preferred_element_type=jnp.float32)
    s = jnp.dot(xw, w12_ref[...], preferred_element_type=jnp.float32)
    xws_ref[0] = xw[:, :128]
    xws_ref[1] = xw[:, 128:]
    s1_ref[...] = s[:, :16]
    s2_ref[...] = s[:, 16:]


def _dense_stage(x, weight, w12p):
    grid = N // _ROW_BLK
    return pl.pallas_call(
        _tc_body,
        grid=(grid,),
        in_specs=[
            pl.BlockSpec((_ROW_BLK, IN_CH), lambda i: (i, 0)),
            pl.BlockSpec((IN_CH, C * F), lambda i: (0, 0)),
            pl.BlockSpec((C * F, 32), lambda i: (0, 0)),
        ],
        out_specs=[
            pl.BlockSpec((2, _ROW_BLK, 128), lambda i: (0, i, 0)),
            pl.BlockSpec((_ROW_BLK, 16), lambda i: (i, 0)),
            pl.BlockSpec((_ROW_BLK, 16), lambda i: (i, 0)),
        ],
        out_shape=[
            jax.ShapeDtypeStruct((2, N, 128), jnp.float32),
            jax.ShapeDtypeStruct((N, 16), jnp.float32),
            jax.ShapeDtypeStruct((N, 16), jnp.float32),
        ],
    )(x, weight, w12p)


_SLC_A = 624
_LAST_A = N - 15 * _SLC_A


def _sca_body(row_r, col_r, ea_r, s1_r, s2_r, z8_r,
              ex_r, ssp_r,
              ssum_sp, rowb, colb, eab, g1, g2, exb):
    c = lax.axis_index("c")
    s = lax.axis_index("s")
    wid = s * NSC + c
    iota = lax.iota(jnp.int32, 16)
    lane8 = iota & 7
    ea_lane = (iota >> 1) & 3
    low_mask = iota < 8

    @pl.when(s < 15)
    def _():
        pltpu.sync_copy(z8_r.at[pl.ds(s * _SLC_A, _SLC_A)],
                        ssum_sp.at[pl.ds(s * _SLC_A, _SLC_A)])

    @pl.when(s == 15)
    def _():
        pltpu.sync_copy(z8_r.at[pl.ds(15 * _SLC_A, _LAST_A)],
                        ssum_sp.at[pl.ds(15 * _SLC_A, _LAST_A)])

    plsc.subcore_barrier()

    def chunk_body(j, _):
        cid = wid + NW * j
        e0 = cid * CH
        pltpu.sync_copy(row_r.at[pl.ds(e0, CH)], rowb)
        pltpu.sync_copy(col_r.at[pl.ds(e0, CH)], colb)
        pltpu.sync_copy(ea_r.at[pl.ds(e0, CH)], eab)
        pltpu.sync_copy(s1_r.at[rowb], g1)
        pltpu.sync_copy(s2_r.at[colb], g2)

        def edge(e, _):
            ev = jnp.broadcast_to(e, (16,))
            t = (plsc.load_gather(g1, [ev, iota])
                 + plsc.load_gather(g2, [ev, iota]))
            sc = plsc.load_gather(eab, [ev, ea_lane])
            a = t * sc
            a = jnp.maximum(a, NEG * a)
            ex = jnp.exp(a)
            plsc.store_scatter(exb, [ev, lane8], ex, mask=low_mask)
            return 0

        lax.fori_loop(0, CH, edge, 0, unroll=4)
        pltpu.sync_copy(exb, ex_r.at[pl.ds(e0, CH)])
        pltpu.sync_copy(exb, ssum_sp.at[rowb], add=True)
        return 0

    nj = jnp.where(wid < NCHUNK - (NCHUNK // NW) * NW,
                   NCHUNK // NW + 1, NCHUNK // NW)
    lax.fori_loop(0, nj, chunk_body, 0)
    plsc.subcore_barrier()

    def flush(r0, nrow):
        pltpu.sync_copy(ssum_sp.at[pl.ds(r0, nrow)],
                        ssp_r.at[c, pl.ds(r0, nrow)])

    @pl.when(s < 15)
    def _():
        flush(s * _SLC_A, _SLC_A)

    @pl.when(s == 15)
    def _():
        flush(15 * _SLC_A, _LAST_A)


_sc_a = functools.partial(
    pl.kernel,
    out_type=[
        jax.ShapeDtypeStruct((E, C), jnp.float32),
        jax.ShapeDtypeStruct((2, N, C), jnp.float32),
    ],
    mesh=_mesh,
    compiler_params=_sc_params,
    scratch_types=[
        pltpu.VMEM_SHARED((N, C), jnp.float32),
        pltpu.VMEM((CH,), jnp.int32),
        pltpu.VMEM((CH,), jnp.int32),
        pltpu.VMEM((CH, 4), jnp.float32),
        pltpu.VMEM((CH, 16), jnp.float32),
        pltpu.VMEM((CH, 16), jnp.float32),
        pltpu.VMEM((CH, 8), jnp.float32),
    ],
)(_sca_body)


_EPI = 125
_NJ = NCHUNK // NT
_NEXTRA = NCHUNK - _NJ * NT


def _scc_body(row_r, col_r, ex_r, ssp_r, xws_r, bias_r, z128_r,
              o_r, ap_r,
              out_sp, rowb, colb, exb, d0b, d1b, gbuf, apb, biasb, semG):
    c = lax.axis_index("c")
    s = lax.axis_index("s")
    iota = lax.iota(jnp.int32, 16)
    lane8 = iota & 7
    lane_e = iota >> 3
    c4 = c * 4
    amask = (lane8 >= c4) & (lane8 < c4 + 4)

    pltpu.sync_copy(z128_r.at[pl.ds(s * 625, 625)],
                    out_sp.at[pl.ds(s * 625, 625)])
    pltpu.sync_copy(bias_r.at[pl.ds(c * 128, 128)], biasb)
    plsc.subcore_barrier()

    def sync_l(j, b):
        e0 = jnp.minimum(s + NT * j, NCHUNK - 1) * CH
        pltpu.sync_copy(row_r.at[pl.ds(e0, CH)], rowb.at[b])
        pltpu.sync_copy(col_r.at[pl.ds(e0, CH)], colb.at[b])
        pltpu.sync_copy(ex_r.at[pl.ds(e0, CH)], exb.at[b])

    def issue_g(b):
        pltpu.async_copy(ssp_r.at[0].at[rowb.at[b]], d0b.at[b], semG.at[b])
        pltpu.async_copy(ssp_r.at[1].at[rowb.at[b]], d1b.at[b], semG.at[b])
        pltpu.async_copy(xws_r.at[c].at[colb.at[b]], gbuf.at[b], semG.at[b])

    def drain_g(b):
        pltpu.make_async_copy(ssp_r.at[0, pl.ds(0, CH)], d0b.at[b],
                              semG.at[b]).wait()
        pltpu.make_async_copy(ssp_r.at[1, pl.ds(0, CH)], d1b.at[b],
                              semG.at[b]).wait()
        pltpu.make_async_copy(xws_r.at[0, pl.ds(0, CH)], gbuf.at[b],
                              semG.at[b]).wait()

    def pair_loop(bv):
        def pair(k, _):
            e = k * 2
            ev = jnp.broadcast_to(e, (16,)) + lane_e
            exv = plsc.load_gather(exb, [bv, ev, lane8])
            d = (plsc.load_gather(d0b, [bv, ev, lane8])
                 + plsc.load_gather(d1b, [bv, ev, lane8]) + 1e-16)
            al = exv / d
            plsc.store_scatter(apb, [bv, ev, lane8 - c4], al, mask=amask)
            return 0

        lax.fori_loop(0, CH // 2, pair, 0, unroll=4)

    def scale_loop(bv):
        def scale(e, _):
            erow = jnp.broadcast_to(e, (16,))
            for h in range(4):
                m = plsc.load_gather(
                    apb, [bv, erow, jnp.broadcast_to(h, (16,))])
                for r in (2 * h, 2 * h + 1):
                    colv = r * 16 + iota
                    g = plsc.load_gather(gbuf, [bv, erow, colv])
                    plsc.store_scatter(gbuf, [bv, erow, colv], g * m)
            return 0

        lax.fori_loop(0, CH, scale, 0, unroll=2)

    sync_l(0, 0)
    issue_g(0)

    def chunk_body(j, _):
        b = j & 1
        b1 = 1 - b
        bv = jnp.broadcast_to(b, (16,))
        e0 = (s + NT * j) * CH
        sync_l(j + 1, b1)
        issue_g(b1)
        drain_g(b)
        pair_loop(bv)
        scale_loop(bv)
        pltpu.sync_copy(apb.at[b], ap_r.at[c, pl.ds(e0, CH)])
        pltpu.sync_copy(gbuf.at[b], out_sp.at[rowb.at[b]], add=True)
        return 0

    lax.fori_loop(0, _NJ, chunk_body, 0)
    drain_g(_NJ & 1)

    zv = jnp.broadcast_to(0, (16,))
    e0x = ((s & 3) + _NJ * NT) * CH
    pltpu.sync_copy(row_r.at[pl.ds(e0x, CH)], rowb.at[0])
    pltpu.sync_copy(col_r.at[pl.ds(e0x, CH)], colb.at[0])
    pltpu.sync_copy(ex_r.at[pl.ds(e0x, CH)], exb.at[0])
    pltpu.sync_copy(ssp_r.at[0].at[rowb.at[0]], d0b.at[0])
    pltpu.sync_copy(ssp_r.at[1].at[rowb.at[0]], d1b.at[0])
    pltpu.sync_copy(xws_r.at[c].at[colb.at[0]], gbuf.at[0])
    pair_loop(zv)
    scale_loop(zv)

    def redirect(v, _):
        lv = v * 16 + iota
        rv = plsc.load_gather(rowb, [zv, lv])
        rv2 = jnp.where(s < _NEXTRA, rv, N)
        plsc.store_scatter(rowb, [zv, lv], rv2)
        return 0

    lax.fori_loop(0, CH // 16, redirect, 0)
    pltpu.sync_copy(apb.at[0], ap_r.at[c, pl.ds(e0x, CH)])
    pltpu.sync_copy(gbuf.at[0], out_sp.at[rowb.at[0]], add=True)

    plsc.subcore_barrier()

    def batch(b, _):
        r0 = s * 625 + b * _EPI
        pltpu.sync_copy(out_sp.at[pl.ds(r0, _EPI)],
                        gbuf.at[0].at[pl.ds(0, _EPI)])

        def browe(m, _):
            mrow = jnp.broadcast_to(m, (16,))
            for r in range(8):
                colv = r * 16 + iota
                g = plsc.load_gather(gbuf, [zv, mrow, colv])
                plsc.store_scatter(gbuf, [zv, mrow, colv],
                                   g + biasb[pl.ds(r * 16, 16)])
            return 0

        lax.fori_loop(0, _EPI, browe, 0, unroll=2)
        pltpu.sync_copy(gbuf.at[0].at[pl.ds(0, _EPI)],
                        o_r.at[c, pl.ds(r0, _EPI)])
        return 0

    lax.fori_loop(0, 5, batch, 0)


_sc_c = functools.partial(
    pl.kernel,
    out_type=[
        jax.ShapeDtypeStruct((2, N, 128), jnp.float32),
        jax.ShapeDtypeStruct((2, E, 4), jnp.float32),
    ],
    mesh=_mesh,
    compiler_params=_sc_params,
    scratch_types=[
        pltpu.VMEM_SHARED((N + 16, 128), jnp.float32),
        pltpu.VMEM((2, CH), jnp.int32),
        pltpu.VMEM((2, CH), jnp.int32),
        pltpu.VMEM((2, CH, 8), jnp.float32),
        pltpu.VMEM((2, CH, 8), jnp.float32),
        pltpu.VMEM((2, CH, 8), jnp.float32),
        pltpu.VMEM((2, CH, 128), jnp.float32),
        pltpu.VMEM((2, CH, 4), jnp.float32),
        pltpu.VMEM((128,), jnp.float32),
        pltpu.SemaphoreType.DMA((2,)),
    ],
)(_scc_body)


def kernel(x, edge_index, edge_attr, weight, att_weight, bias):
    aw1 = att_weight[0, :, :F]
    aw2 = att_weight[0, :, F:]
    eye = jnp.eye(C, dtype=jnp.float32)
    w1 = (aw1[:, :, None] * eye[:, None, :]).reshape(C * F, C)
    w2 = (aw2[:, :, None] * eye[:, None, :]).reshape(C * F, C)
    zc = jnp.zeros((C * F, C), jnp.float32)
    w12p = jnp.concatenate([w1, zc, w2, zc], axis=1)

    xws, s1p, s2p = _dense_stage(x, weight, w12p)

    row = edge_index[0]
    col = edge_index[1]
    z8 = jnp.zeros((N, C), jnp.float32)
    z128 = jnp.zeros((N, 128), jnp.float32)

    ex, ssp = _sc_a(row, col, edge_attr, s1p, s2p, z8)
    o2, ap2 = _sc_c(row, col, ex, ssp, xws, bias, z128)

    out = jnp.concatenate([o2[0], o2[1]], axis=1)
    alpha = jnp.concatenate([ap2[0], ap2[1]], axis=1)
    return (out, alpha)


# speedup vs baseline: 1.2043x; 1.0433x over previous
import functools

import jax
import jax.numpy as jnp
from jax import lax
from jax.experimental import pallas as pl
from jax.experimental.pallas import tpu as pltpu
from jax.experimental.pallas import tpu_sc as plsc

N = 10000
E = 320000
IN_CH = 128
OUT_CH = 32
HEADS = 2
EAD = 4
C = HEADS * EAD
F = OUT_CH
NEG = 0.2

CH = 128
NCHUNK = E // CH
NSC = 2
NT = 16
NW = NSC * NT

_ROW_BLK = 1000

_mesh = plsc.VectorSubcoreMesh(core_axis_name="c", subcore_axis_name="s")
_sc_params = pltpu.CompilerParams(needs_layout_passes=False,
                                  use_tc_tiling_on_sc=False)


def _tc_body_s(x_ref, w_ref, w12_ref, s1_ref, s2_ref):
    xw = jnp.dot(x_ref[...], w_ref[...], preferred_element_type=jnp.float32)
    s = jnp.dot(xw, w12_ref[...], preferred_element_type=jnp.float32)
    s1_ref[...] = s[:, :16]
    s2_ref[...] = s[:, 16:]


def _tc_body_x(x_ref, w_ref, xws_ref):
    xw = jnp.dot(x_ref[...], w_ref[...], preferred_element_type=jnp.float32)
    xws_ref[0] = xw[:, :128]
    xws_ref[1] = xw[:, 128:]


def _dense_stage_s(x, weight, w12p):
    grid = N // _ROW_BLK
    return pl.pallas_call(
        _tc_body_s,
        grid=(grid,),
        in_specs=[
            pl.BlockSpec((_ROW_BLK, IN_CH), lambda i: (i, 0)),
            pl.BlockSpec((IN_CH, C * F), lambda i: (0, 0)),
            pl.BlockSpec((C * F, 32), lambda i: (0, 0)),
        ],
        out_specs=[
            pl.BlockSpec((_ROW_BLK, 16), lambda i: (i, 0)),
            pl.BlockSpec((_ROW_BLK, 16), lambda i: (i, 0)),
        ],
        out_shape=[
            jax.ShapeDtypeStruct((N, 16), jnp.float32),
            jax.ShapeDtypeStruct((N, 16), jnp.float32),
        ],
    )(x, weight, w12p)


def _dense_stage_x(x, weight):
    grid = N // _ROW_BLK
    return pl.pallas_call(
        _tc_body_x,
        grid=(grid,),
        in_specs=[
            pl.BlockSpec((_ROW_BLK, IN_CH), lambda i: (i, 0)),
            pl.BlockSpec((IN_CH, C * F), lambda i: (0, 0)),
        ],
        out_specs=[
            pl.BlockSpec((2, _ROW_BLK, 128), lambda i: (0, i, 0)),
        ],
        out_shape=[
            jax.ShapeDtypeStruct((2, N, 128), jnp.float32),
        ],
    )(x, weight)


_SLC_A = 624
_LAST_A = N - 15 * _SLC_A
_NJA = NCHUNK // NW
_NEXTRA_A = NCHUNK - _NJA * NW


def _sca_body(row_r, col_r, ea_r, s1_r, s2_r, z8_r,
              ex_r, ssp_r,
              ssum_sp, rowb, colb, eab, g1, g2, exb, semG):
    c = lax.axis_index("c")
    s = lax.axis_index("s")
    wid = s * NSC + c
    iota = lax.iota(jnp.int32, 16)
    lane8 = iota & 7
    ea_lane = (iota >> 1) & 3
    low_mask = iota < 8

    @pl.when(s < 15)
    def _():
        pltpu.sync_copy(z8_r.at[pl.ds(s * _SLC_A, _SLC_A)],
                        ssum_sp.at[pl.ds(s * _SLC_A, _SLC_A)])

    @pl.when(s == 15)
    def _():
        pltpu.sync_copy(z8_r.at[pl.ds(15 * _SLC_A, _LAST_A + 16)],
                        ssum_sp.at[pl.ds(15 * _SLC_A, _LAST_A + 16)])

    plsc.subcore_barrier()

    def sync_l(j, b):
        e0 = jnp.minimum(wid + NW * j, NCHUNK - 1) * CH
        pltpu.sync_copy(row_r.at[pl.ds(e0, CH)], rowb.at[b])
        pltpu.sync_copy(col_r.at[pl.ds(e0, CH)], colb.at[b])
        pltpu.sync_copy(ea_r.at[pl.ds(e0, CH)], eab.at[b])

    def issue_g(b):
        pltpu.async_copy(s1_r.at[rowb.at[b]], g1.at[b], semG.at[b])
        pltpu.async_copy(s2_r.at[colb.at[b]], g2.at[b], semG.at[b])

    def drain_g(b):
        pltpu.make_async_copy(s1_r.at[pl.ds(0, CH)], g1.at[b],
                              semG.at[b]).wait()
        pltpu.make_async_copy(s2_r.at[pl.ds(0, CH)], g2.at[b],
                              semG.at[b]).wait()

    def edge_loop(bv):
        def edge(e, _):
            ev = jnp.broadcast_to(e, (16,))
            t = (plsc.load_gather(g1, [bv, ev, iota])
                 + plsc.load_gather(g2, [bv, ev, iota]))
            sc = plsc.load_gather(eab, [bv, ev, ea_lane])
            a = t * sc
            a = jnp.maximum(a, NEG * a)
            ex = jnp.exp(a)
            plsc.store_scatter(exb, [ev, lane8], ex, mask=low_mask)
            return 0

        lax.fori_loop(0, CH, edge, 0, unroll=4)

    sync_l(0, 0)
    issue_g(0)

    def chunk_body(j, _):
        b = j & 1
        b1 = 1 - b
        bv = jnp.broadcast_to(b, (16,))
        e0 = (wid + NW * j) * CH
        sync_l(j + 1, b1)
        issue_g(b1)
        drain_g(b)
        edge_loop(bv)
        pltpu.sync_copy(exb, ex_r.at[pl.ds(e0, CH)])
        pltpu.sync_copy(exb, ssum_sp.at[rowb.at[b]], add=True)
        return 0

    lax.fori_loop(0, _NJA, chunk_body, 0)
    drain_g(_NJA & 1)

    zv = jnp.broadcast_to(0, (16,))
    cidx = (wid & 3) + _NJA * NW
    e0x = cidx * CH
    pltpu.sync_copy(row_r.at[pl.ds(e0x, CH)], rowb.at[0])
    pltpu.sync_copy(col_r.at[pl.ds(e0x, CH)], colb.at[0])
    pltpu.sync_copy(ea_r.at[pl.ds(e0x, CH)], eab.at[0])
    pltpu.sync_copy(s1_r.at[rowb.at[0]], g1.at[0])
    pltpu.sync_copy(s2_r.at[colb.at[0]], g2.at[0])
    edge_loop(zv)

    def redirect(v, _):
        lv = v * 16 + iota
        rv = plsc.load_gather(rowb, [zv, lv])
        rv2 = jnp.where(wid < _NEXTRA_A, rv, N)
        plsc.store_scatter(rowb, [zv, lv], rv2)
        return 0

    lax.fori_loop(0, CH // 16, redirect, 0)
    pltpu.sync_copy(exb, ex_r.at[pl.ds(e0x, CH)])
    pltpu.sync_copy(exb, ssum_sp.at[rowb.at[0]], add=True)

    plsc.subcore_barrier()

    def flush(r0, nrow):
        pltpu.sync_copy(ssum_sp.at[pl.ds(r0, nrow)],
                        ssp_r.at[c, pl.ds(r0, nrow)])

    @pl.when(s < 15)
    def _():
        flush(s * _SLC_A, _SLC_A)

    @pl.when(s == 15)
    def _():
        flush(15 * _SLC_A, _LAST_A)


_sc_a = functools.partial(
    pl.kernel,
    out_type=[
        jax.ShapeDtypeStruct((E, C), jnp.float32),
        jax.ShapeDtypeStruct((2, N, C), jnp.float32),
    ],
    mesh=_mesh,
    compiler_params=_sc_params,
    scratch_types=[
        pltpu.VMEM_SHARED((N + 16, C), jnp.float32),
        pltpu.VMEM((2, CH), jnp.int32),
        pltpu.VMEM((2, CH), jnp.int32),
        pltpu.VMEM((2, CH, 4), jnp.float32),
        pltpu.VMEM((2, CH, 16), jnp.float32),
        pltpu.VMEM((2, CH, 16), jnp.float32),
        pltpu.VMEM((CH, 8), jnp.float32),
        pltpu.SemaphoreType.DMA((2,)),
    ],
)(_sca_body)


_EPI = 125
_NJ = NCHUNK // NT
_NEXTRA = NCHUNK - _NJ * NT


def _scc_body(row_r, col_r, ex_r, ssp_r, xws_r, bias_r, z128_r,
              o_r, ap_r,
              out_sp, rowb, colb, exb, d0b, d1b, gbuf, apb, biasb, semG):
    c = lax.axis_index("c")
    s = lax.axis_index("s")
    iota = lax.iota(jnp.int32, 16)
    lane8 = iota & 7
    lane_e = iota >> 3
    c4 = c * 4
    amask = (lane8 >= c4) & (lane8 < c4 + 4)

    pltpu.sync_copy(z128_r.at[pl.ds(s * 625, 625)],
                    out_sp.at[pl.ds(s * 625, 625)])
    pltpu.sync_copy(bias_r.at[pl.ds(c * 128, 128)], biasb)
    plsc.subcore_barrier()

    def sync_l(j, b):
        e0 = jnp.minimum(s + NT * j, NCHUNK - 1) * CH
        pltpu.sync_copy(row_r.at[pl.ds(e0, CH)], rowb.at[b])
        pltpu.sync_copy(col_r.at[pl.ds(e0, CH)], colb.at[b])
        pltpu.sync_copy(ex_r.at[pl.ds(e0, CH)], exb.at[b])

    def issue_g(b):
        pltpu.async_copy(ssp_r.at[0].at[rowb.at[b]], d0b.at[b], semG.at[b])
        pltpu.async_copy(ssp_r.at[1].at[rowb.at[b]], d1b.at[b], semG.at[b])
        pltpu.async_copy(xws_r.at[c].at[colb.at[b]], gbuf.at[b], semG.at[b])

    def drain_g(b):
        pltpu.make_async_copy(ssp_r.at[0, pl.ds(0, CH)], d0b.at[b],
                              semG.at[b]).wait()
        pltpu.make_async_copy(ssp_r.at[1, pl.ds(0, CH)], d1b.at[b],
                              semG.at[b]).wait()
        pltpu.make_async_copy(xws_r.at[0, pl.ds(0, CH)], gbuf.at[b],
                              semG.at[b]).wait()

    def pair_loop(bv):
        def pair(k, _):
            e = k * 2
            ev = jnp.broadcast_to(e, (16,)) + lane_e
            exv = plsc.load_gather(exb, [bv, ev, lane8])
            d = (plsc.load_gather(d0b, [bv, ev, lane8])
                 + plsc.load_gather(d1b, [bv, ev, lane8]) + 1e-16)
            al = exv / d
            plsc.store_scatter(apb, [bv, ev, lane8 - c4], al, mask=amask)
            return 0

        lax.fori_loop(0, CH // 2, pair, 0, unroll=4)

    def scale_loop(bv):
        def scale(e, _):
            erow = jnp.broadcast_to(e, (16,))
            for h in range(4):
                m = plsc.load_gather(
                    apb, [bv, erow, jnp.broadcast_to(h, (16,))])
                for r in (2 * h, 2 * h + 1):
                    colv = r * 16 + iota
                    g = plsc.load_gather(gbuf, [bv, erow, colv])
                    plsc.store_scatter(gbuf, [bv, erow, colv], g * m)
            return 0

        lax.fori_loop(0, CH, scale, 0, unroll=2)

    sync_l(0, 0)
    issue_g(0)

    def chunk_body(j, _):
        b = j & 1
        b1 = 1 - b
        bv = jnp.broadcast_to(b, (16,))
        e0 = (s + NT * j) * CH
        sync_l(j + 1, b1)
        issue_g(b1)
        drain_g(b)
        pair_loop(bv)
        scale_loop(bv)
        pltpu.sync_copy(apb.at[b], ap_r.at[c, pl.ds(e0, CH)])
        pltpu.sync_copy(gbuf.at[b], out_sp.at[rowb.at[b]], add=True)
        return 0

    lax.fori_loop(0, _NJ, chunk_body, 0)
    drain_g(_NJ & 1)

    zv = jnp.broadcast_to(0, (16,))
    e0x = ((s & 3) + _NJ * NT) * CH
    pltpu.sync_copy(row_r.at[pl.ds(e0x, CH)], rowb.at[0])
    pltpu.sync_copy(col_r.at[pl.ds(e0x, CH)], colb.at[0])
    pltpu.sync_copy(ex_r.at[pl.ds(e0x, CH)], exb.at[0])
    pltpu.sync_copy(ssp_r.at[0].at[rowb.at[0]], d0b.at[0])
    pltpu.sync_copy(ssp_r.at[1].at[rowb.at[0]], d1b.at[0])
    pltpu.sync_copy(xws_r.at[c].at[colb.at[0]], gbuf.at[0])
    pair_loop(zv)
    scale_loop(zv)

    def redirect(v, _):
        lv = v * 16 + iota
        rv = plsc.load_gather(rowb, [zv, lv])
        rv2 = jnp.where(s < _NEXTRA, rv, N)
        plsc.store_scatter(rowb, [zv, lv], rv2)
        return 0

    lax.fori_loop(0, CH // 16, redirect, 0)
    pltpu.sync_copy(apb.at[0], ap_r.at[c, pl.ds(e0x, CH)])
    pltpu.sync_copy(gbuf.at[0], out_sp.at[rowb.at[0]], add=True)

    plsc.subcore_barrier()

    def batch(b, _):
        r0 = s * 625 + b * _EPI
        pltpu.sync_copy(out_sp.at[pl.ds(r0, _EPI)],
                        gbuf.at[0].at[pl.ds(0, _EPI)])

        def browe(m, _):
            mrow = jnp.broadcast_to(m, (16,))
            for r in range(8):
                colv = r * 16 + iota
                g = plsc.load_gather(gbuf, [zv, mrow, colv])
                plsc.store_scatter(gbuf, [zv, mrow, colv],
                                   g + biasb[pl.ds(r * 16, 16)])
            return 0

        lax.fori_loop(0, _EPI, browe, 0, unroll=2)
        pltpu.sync_copy(gbuf.at[0].at[pl.ds(0, _EPI)],
                        o_r.at[c, pl.ds(r0, _EPI)])
        return 0

    lax.fori_loop(0, 5, batch, 0)


_sc_c = functools.partial(
    pl.kernel,
    out_type=[
        jax.ShapeDtypeStruct((2, N, 128), jnp.float32),
        jax.ShapeDtypeStruct((2, E, 4), jnp.float32),
    ],
    mesh=_mesh,
    compiler_params=_sc_params,
    scratch_types=[
        pltpu.VMEM_SHARED((N + 16, 128), jnp.float32),
        pltpu.VMEM((2, CH), jnp.int32),
        pltpu.VMEM((2, CH), jnp.int32),
        pltpu.VMEM((2, CH, 8), jnp.float32),
        pltpu.VMEM((2, CH, 8), jnp.float32),
        pltpu.VMEM((2, CH, 8), jnp.float32),
        pltpu.VMEM((2, CH, 128), jnp.float32),
        pltpu.VMEM((2, CH, 4), jnp.float32),
        pltpu.VMEM((128,), jnp.float32),
        pltpu.SemaphoreType.DMA((2,)),
    ],
)(_scc_body)


def kernel(x, edge_index, edge_attr, weight, att_weight, bias):
    aw1 = att_weight[0, :, :F]
    aw2 = att_weight[0, :, F:]
    eye = jnp.eye(C, dtype=jnp.float32)
    w1 = (aw1[:, :, None] * eye[:, None, :]).reshape(C * F, C)
    w2 = (aw2[:, :, None] * eye[:, None, :]).reshape(C * F, C)
    zc = jnp.zeros((C * F, C), jnp.float32)
    w12p = jnp.concatenate([w1, zc, w2, zc], axis=1)

    s1p, s2p = _dense_stage_s(x, weight, w12p)
    (xws,) = _dense_stage_x(x, weight)

    row = edge_index[0]
    col = edge_index[1]
    z8 = jnp.zeros((N + 16, C), jnp.float32)
    z128 = jnp.zeros((N, 128), jnp.float32)

    ex, ssp = _sc_a(row, col, edge_attr, s1p, s2p, z8)
    o2, ap2 = _sc_c(row, col, ex, ssp, xws, bias, z128)

    out = jnp.concatenate([o2[0], o2[1]], axis=1)
    alpha = jnp.concatenate([ap2[0], ap2[1]], axis=1)
    return (out, alpha)


# speedup vs baseline: 1.4285x; 1.1861x over previous
import functools

import jax
import jax.numpy as jnp
from jax import lax
from jax.experimental import pallas as pl
from jax.experimental.pallas import tpu as pltpu
from jax.experimental.pallas import tpu_sc as plsc

N = 10000
E = 320000
IN_CH = 128
OUT_CH = 32
HEADS = 2
EAD = 4
C = HEADS * EAD
F = OUT_CH
NEG = 0.2

CH = 128
NCHUNK = E // CH
NSC = 2
NT = 16
NW = NSC * NT

_ROW_BLK = 1000

_mesh = plsc.VectorSubcoreMesh(core_axis_name="c", subcore_axis_name="s")
_sc_params = pltpu.CompilerParams(needs_layout_passes=False,
                                  use_tc_tiling_on_sc=False)


def _tc_body_s(x_ref, w_ref, w12_ref, s1_ref, s2_ref):
    xw = jnp.dot(x_ref[...], w_ref[...], preferred_element_type=jnp.float32)
    s = jnp.dot(xw, w12_ref[...], preferred_element_type=jnp.float32)
    s1_ref[...] = s[:, :16]
    s2_ref[...] = s[:, 16:]


def _tc_body_x(x_ref, w_ref, xws_ref):
    xw = jnp.dot(x_ref[...], w_ref[...], preferred_element_type=jnp.float32)
    xws_ref[0] = xw[:, :128]
    xws_ref[1] = xw[:, 128:]


def _dense_stage_s(x, weight, w12p):
    grid = N // _ROW_BLK
    return pl.pallas_call(
        _tc_body_s,
        grid=(grid,),
        in_specs=[
            pl.BlockSpec((_ROW_BLK, IN_CH), lambda i: (i, 0)),
            pl.BlockSpec((IN_CH, C * F), lambda i: (0, 0)),
            pl.BlockSpec((C * F, 32), lambda i: (0, 0)),
        ],
        out_specs=[
            pl.BlockSpec((_ROW_BLK, 16), lambda i: (i, 0)),
            pl.BlockSpec((_ROW_BLK, 16), lambda i: (i, 0)),
        ],
        out_shape=[
            jax.ShapeDtypeStruct((N, 16), jnp.float32),
            jax.ShapeDtypeStruct((N, 16), jnp.float32),
        ],
    )(x, weight, w12p)


def _dense_stage_x(x, weight):
    grid = N // _ROW_BLK
    return pl.pallas_call(
        _tc_body_x,
        grid=(grid,),
        in_specs=[
            pl.BlockSpec((_ROW_BLK, IN_CH), lambda i: (i, 0)),
            pl.BlockSpec((IN_CH, C * F), lambda i: (0, 0)),
        ],
        out_specs=[
            pl.BlockSpec((2, _ROW_BLK, 128), lambda i: (0, i, 0)),
        ],
        out_shape=[
            jax.ShapeDtypeStruct((2, N, 128), jnp.float32),
        ],
    )(x, weight)


_SLC_A = 624
_LAST_A = N - 15 * _SLC_A
_NJA = NCHUNK // NW
_NEXTRA_A = NCHUNK - _NJA * NW


def _sca_body(row_r, col_r, ea_r, s1_r, s2_r, z8_r,
              ex_r, ssp_r,
              ssum_sp, rowb, colb, eab, g1, g2, exb, semG):
    c = lax.axis_index("c")
    s = lax.axis_index("s")
    wid = s * NSC + c
    iota = lax.iota(jnp.int32, 16)
    lane8 = iota & 7
    ea_lane = (iota >> 1) & 3
    low_mask = iota < 8

    @pl.when(s < 15)
    def _():
        pltpu.sync_copy(z8_r.at[pl.ds(s * _SLC_A, _SLC_A)],
                        ssum_sp.at[pl.ds(s * _SLC_A, _SLC_A)])

    @pl.when(s == 15)
    def _():
        pltpu.sync_copy(z8_r.at[pl.ds(15 * _SLC_A, _LAST_A + 16)],
                        ssum_sp.at[pl.ds(15 * _SLC_A, _LAST_A + 16)])

    plsc.subcore_barrier()

    def sync_l(j, b):
        e0 = jnp.minimum(wid + NW * j, NCHUNK - 1) * CH
        pltpu.sync_copy(row_r.at[pl.ds(e0, CH)], rowb.at[b])
        pltpu.sync_copy(col_r.at[pl.ds(e0, CH)], colb.at[b])
        pltpu.sync_copy(ea_r.at[pl.ds(e0, CH)], eab.at[b])

    def issue_g(b):
        pltpu.async_copy(s1_r.at[rowb.at[b]], g1.at[b], semG.at[b])
        pltpu.async_copy(s2_r.at[colb.at[b]], g2.at[b], semG.at[b])

    def drain_g(b):
        pltpu.make_async_copy(s1_r.at[pl.ds(0, CH)], g1.at[b],
                              semG.at[b]).wait()
        pltpu.make_async_copy(s2_r.at[pl.ds(0, CH)], g2.at[b],
                              semG.at[b]).wait()

    def edge_loop(bs, bv):
        def edge(e, _):
            ev = jnp.broadcast_to(e, (16,))
            t = g1[bs, e] + g2[bs, e]
            sc = plsc.load_gather(eab, [bv, ev, ea_lane])
            a = t * sc
            a = jnp.maximum(a, NEG * a)
            ex = jnp.exp(a)
            plsc.store_scatter(exb, [ev, lane8], ex, mask=low_mask)
            return 0

        lax.fori_loop(0, CH, edge, 0, unroll=4)

    sync_l(0, 0)
    issue_g(0)

    def chunk_body(j, _):
        b = j & 1
        b1 = 1 - b
        bv = jnp.broadcast_to(b, (16,))
        e0 = (wid + NW * j) * CH
        sync_l(j + 1, b1)
        issue_g(b1)
        drain_g(b)
        edge_loop(b, bv)
        pltpu.sync_copy(exb, ex_r.at[pl.ds(e0, CH)])
        pltpu.sync_copy(exb, ssum_sp.at[rowb.at[b]], add=True)
        return 0

    lax.fori_loop(0, _NJA, chunk_body, 0)
    drain_g(_NJA & 1)

    zv = jnp.broadcast_to(0, (16,))
    cidx = (wid & 3) + _NJA * NW
    e0x = cidx * CH
    pltpu.sync_copy(row_r.at[pl.ds(e0x, CH)], rowb.at[0])
    pltpu.sync_copy(col_r.at[pl.ds(e0x, CH)], colb.at[0])
    pltpu.sync_copy(ea_r.at[pl.ds(e0x, CH)], eab.at[0])
    pltpu.sync_copy(s1_r.at[rowb.at[0]], g1.at[0])
    pltpu.sync_copy(s2_r.at[colb.at[0]], g2.at[0])
    edge_loop(0, zv)

    def redirect(v, _):
        lv = v * 16 + iota
        rv = plsc.load_gather(rowb, [zv, lv])
        rv2 = jnp.where(wid < _NEXTRA_A, rv, N)
        plsc.store_scatter(rowb, [zv, lv], rv2)
        return 0

    lax.fori_loop(0, CH // 16, redirect, 0)
    pltpu.sync_copy(exb, ex_r.at[pl.ds(e0x, CH)])
    pltpu.sync_copy(exb, ssum_sp.at[rowb.at[0]], add=True)

    plsc.subcore_barrier()

    def flush(r0, nrow):
        pltpu.sync_copy(ssum_sp.at[pl.ds(r0, nrow)],
                        ssp_r.at[c, pl.ds(r0, nrow)])

    @pl.when(s < 15)
    def _():
        flush(s * _SLC_A, _SLC_A)

    @pl.when(s == 15)
    def _():
        flush(15 * _SLC_A, _LAST_A)


_sc_a = functools.partial(
    pl.kernel,
    out_type=[
        jax.ShapeDtypeStruct((E, C), jnp.float32),
        jax.ShapeDtypeStruct((2, N, C), jnp.float32),
    ],
    mesh=_mesh,
    compiler_params=_sc_params,
    scratch_types=[
        pltpu.VMEM_SHARED((N + 16, C), jnp.float32),
        pltpu.VMEM((2, CH), jnp.int32),
        pltpu.VMEM((2, CH), jnp.int32),
        pltpu.VMEM((2, CH, 4), jnp.float32),
        pltpu.VMEM((2, CH, 16), jnp.float32),
        pltpu.VMEM((2, CH, 16), jnp.float32),
        pltpu.VMEM((CH, 8), jnp.float32),
        pltpu.SemaphoreType.DMA((2,)),
    ],
)(_sca_body)


_EPI = 125
_NJ = NCHUNK // NT
_NEXTRA = NCHUNK - _NJ * NT


def _scc_body(row_r, col_r, ex_r, ssp_r, xws_r, bias_r, z128_r,
              o_r, ap_r,
              out_sp, rowb, colb, exb, d0b, d1b, gbuf, apb, biasb, semG):
    c = lax.axis_index("c")
    s = lax.axis_index("s")
    iota = lax.iota(jnp.int32, 16)
    lane8 = iota & 7
    lane_e = iota >> 3
    c4 = c * 4
    amask = (lane8 >= c4) & (lane8 < c4 + 4)

    pltpu.sync_copy(z128_r.at[pl.ds(s * 625, 625)],
                    out_sp.at[pl.ds(s * 625, 625)])
    pltpu.sync_copy(bias_r.at[pl.ds(c * 128, 128)], biasb)
    plsc.subcore_barrier()

    def sync_l(j, b):
        e0 = jnp.minimum(s + NT * j, NCHUNK - 1) * CH
        pltpu.sync_copy(row_r.at[pl.ds(e0, CH)], rowb.at[b])
        pltpu.sync_copy(col_r.at[pl.ds(e0, CH)], colb.at[b])
        pltpu.sync_copy(ex_r.at[pl.ds(e0, CH)], exb.at[b])

    def issue_g(b):
        pltpu.async_copy(ssp_r.at[0].at[rowb.at[b]], d0b.at[b], semG.at[b])
        pltpu.async_copy(ssp_r.at[1].at[rowb.at[b]], d1b.at[b], semG.at[b])
        pltpu.async_copy(xws_r.at[c].at[colb.at[b]], gbuf.at[b], semG.at[b])

    def drain_g(b):
        pltpu.make_async_copy(ssp_r.at[0, pl.ds(0, CH)], d0b.at[b],
                              semG.at[b]).wait()
        pltpu.make_async_copy(ssp_r.at[1, pl.ds(0, CH)], d1b.at[b],
                              semG.at[b]).wait()
        pltpu.make_async_copy(xws_r.at[0, pl.ds(0, CH)], gbuf.at[b],
                              semG.at[b]).wait()

    def pair_loop(bv):
        def pair(k, _):
            e = k * 2
            ev = jnp.broadcast_to(e, (16,)) + lane_e
            exv = plsc.load_gather(exb, [bv, ev, lane8])
            d = (plsc.load_gather(d0b, [bv, ev, lane8])
                 + plsc.load_gather(d1b, [bv, ev, lane8]) + 1e-16)
            al = exv / d
            plsc.store_scatter(apb, [bv, ev, lane8 - c4], al, mask=amask)
            return 0

        lax.fori_loop(0, CH // 2, pair, 0, unroll=4)

    def scale_loop(bs, bv):
        def scale(e, _):
            erow = jnp.broadcast_to(e, (16,))
            for h in range(4):
                m = plsc.load_gather(
                    apb, [bv, erow, jnp.broadcast_to(h, (16,))])
                for r in (2 * h, 2 * h + 1):
                    sl = pl.ds(r * 16, 16)
                    gbuf[bs, e, sl] = gbuf[bs, e, sl] * m
            return 0

        lax.fori_loop(0, CH, scale, 0, unroll=2)

    sync_l(0, 0)
    issue_g(0)

    def chunk_body(j, _):
        b = j & 1
        b1 = 1 - b
        bv = jnp.broadcast_to(b, (16,))
        e0 = (s + NT * j) * CH
        sync_l(j + 1, b1)
        issue_g(b1)
        drain_g(b)
        pair_loop(bv)
        scale_loop(b, bv)
        pltpu.sync_copy(apb.at[b], ap_r.at[c, pl.ds(e0, CH)])
        pltpu.sync_copy(gbuf.at[b], out_sp.at[rowb.at[b]], add=True)
        return 0

    lax.fori_loop(0, _NJ, chunk_body, 0)
    drain_g(_NJ & 1)

    zv = jnp.broadcast_to(0, (16,))
    e0x = ((s & 3) + _NJ * NT) * CH
    pltpu.sync_copy(row_r.at[pl.ds(e0x, CH)], rowb.at[0])
    pltpu.sync_copy(col_r.at[pl.ds(e0x, CH)], colb.at[0])
    pltpu.sync_copy(ex_r.at[pl.ds(e0x, CH)], exb.at[0])
    pltpu.sync_copy(ssp_r.at[0].at[rowb.at[0]], d0b.at[0])
    pltpu.sync_copy(ssp_r.at[1].at[rowb.at[0]], d1b.at[0])
    pltpu.sync_copy(xws_r.at[c].at[colb.at[0]], gbuf.at[0])
    pair_loop(zv)
    scale_loop(0, zv)

    def redirect(v, _):
        lv = v * 16 + iota
        rv = plsc.load_gather(rowb, [zv, lv])
        rv2 = jnp.where(s < _NEXTRA, rv, N)
        plsc.store_scatter(rowb, [zv, lv], rv2)
        return 0

    lax.fori_loop(0, CH // 16, redirect, 0)
    pltpu.sync_copy(apb.at[0], ap_r.at[c, pl.ds(e0x, CH)])
    pltpu.sync_copy(gbuf.at[0], out_sp.at[rowb.at[0]], add=True)

    plsc.subcore_barrier()

    def batch(b, _):
        r0 = s * 625 + b * _EPI
        pltpu.sync_copy(out_sp.at[pl.ds(r0, _EPI)],
                        gbuf.at[0].at[pl.ds(0, _EPI)])

        def browe(m, _):
            for r in range(8):
                sl = pl.ds(r * 16, 16)
                gbuf[0, m, sl] = gbuf[0, m, sl] + biasb[pl.ds(r * 16, 16)]
            return 0

        lax.fori_loop(0, _EPI, browe, 0, unroll=2)
        pltpu.sync_copy(gbuf.at[0].at[pl.ds(0, _EPI)],
                        o_r.at[c, pl.ds(r0, _EPI)])
        return 0

    lax.fori_loop(0, 5, batch, 0)


_sc_c = functools.partial(
    pl.kernel,
    out_type=[
        jax.ShapeDtypeStruct((2, N, 128), jnp.float32),
        jax.ShapeDtypeStruct((2, E, 4), jnp.float32),
    ],
    mesh=_mesh,
    compiler_params=_sc_params,
    scratch_types=[
        pltpu.VMEM_SHARED((N + 16, 128), jnp.float32),
        pltpu.VMEM((2, CH), jnp.int32),
        pltpu.VMEM((2, CH), jnp.int32),
        pltpu.VMEM((2, CH, 8), jnp.float32),
        pltpu.VMEM((2, CH, 8), jnp.float32),
        pltpu.VMEM((2, CH, 8), jnp.float32),
        pltpu.VMEM((2, CH, 128), jnp.float32),
        pltpu.VMEM((2, CH, 4), jnp.float32),
        pltpu.VMEM((128,), jnp.float32),
        pltpu.SemaphoreType.DMA((2,)),
    ],
)(_scc_body)


def kernel(x, edge_index, edge_attr, weight, att_weight, bias):
    aw1 = att_weight[0, :, :F]
    aw2 = att_weight[0, :, F:]
    eye = jnp.eye(C, dtype=jnp.float32)
    w1 = (aw1[:, :, None] * eye[:, None, :]).reshape(C * F, C)
    w2 = (aw2[:, :, None] * eye[:, None, :]).reshape(C * F, C)
    zc = jnp.zeros((C * F, C), jnp.float32)
    w12p = jnp.concatenate([w1, zc, w2, zc], axis=1)

    s1p, s2p = _dense_stage_s(x, weight, w12p)
    (xws,) = _dense_stage_x(x, weight)

    row = edge_index[0]
    col = edge_index[1]
    z8 = jnp.zeros((N + 16, C), jnp.float32)
    z128 = jnp.zeros((N, 128), jnp.float32)

    ex, ssp = _sc_a(row, col, edge_attr, s1p, s2p, z8)
    o2, ap2 = _sc_c(row, col, ex, ssp, xws, bias, z128)

    out = jnp.concatenate([o2[0], o2[1]], axis=1)
    alpha = jnp.concatenate([ap2[0], ap2[1]], axis=1)
    return (out, alpha)


# speedup vs baseline: 1.4324x; 1.0027x over previous
import functools

import jax
import jax.numpy as jnp
from jax import lax
from jax.experimental import pallas as pl
from jax.experimental.pallas import tpu as pltpu
from jax.experimental.pallas import tpu_sc as plsc

N = 10000
E = 320000
IN_CH = 128
OUT_CH = 32
HEADS = 2
EAD = 4
C = HEADS * EAD
F = OUT_CH
NEG = 0.2

CH = 128
NCHUNK = E // CH
NSC = 2
NT = 16
NW = NSC * NT

_ROW_BLK = 1000

_mesh = plsc.VectorSubcoreMesh(core_axis_name="c", subcore_axis_name="s")
_sc_params = pltpu.CompilerParams(needs_layout_passes=False,
                                  use_tc_tiling_on_sc=False)


def _tc_body_s(x_ref, w_ref, w12_ref, s1_ref, s2_ref):
    xw = jnp.dot(x_ref[...], w_ref[...], preferred_element_type=jnp.float32)
    s = jnp.dot(xw, w12_ref[...], preferred_element_type=jnp.float32)
    s1_ref[...] = s[:, :16]
    s2_ref[...] = s[:, 16:]


def _tc_body_x(x_ref, w_ref, xws_ref):
    xw = jnp.dot(x_ref[...], w_ref[...], preferred_element_type=jnp.float32)
    xws_ref[0] = xw[:, :128]
    xws_ref[1] = xw[:, 128:]


def _dense_stage_s(x, weight, w12p):
    grid = N // _ROW_BLK
    return pl.pallas_call(
        _tc_body_s,
        grid=(grid,),
        in_specs=[
            pl.BlockSpec((_ROW_BLK, IN_CH), lambda i: (i, 0)),
            pl.BlockSpec((IN_CH, C * F), lambda i: (0, 0)),
            pl.BlockSpec((C * F, 32), lambda i: (0, 0)),
        ],
        out_specs=[
            pl.BlockSpec((_ROW_BLK, 16), lambda i: (i, 0)),
            pl.BlockSpec((_ROW_BLK, 16), lambda i: (i, 0)),
        ],
        out_shape=[
            jax.ShapeDtypeStruct((N, 16), jnp.float32),
            jax.ShapeDtypeStruct((N, 16), jnp.float32),
        ],
    )(x, weight, w12p)


def _dense_stage_x(x, weight):
    grid = N // _ROW_BLK
    return pl.pallas_call(
        _tc_body_x,
        grid=(grid,),
        in_specs=[
            pl.BlockSpec((_ROW_BLK, IN_CH), lambda i: (i, 0)),
            pl.BlockSpec((IN_CH, C * F), lambda i: (0, 0)),
        ],
        out_specs=[
            pl.BlockSpec((2, _ROW_BLK, 128), lambda i: (0, i, 0)),
        ],
        out_shape=[
            jax.ShapeDtypeStruct((2, N, 128), jnp.float32),
        ],
    )(x, weight)


_SLC_A = 624
_LAST_A = N - 15 * _SLC_A
_NJA = NCHUNK // NW
_NEXTRA_A = NCHUNK - _NJA * NW


def _sca_body(row_r, col_r, ea_r, s1_r, s2_r, z8_r,
              ex_r, ssp_r,
              ssum_sp, rowb, colb, eab, g1, g2, exb, semG):
    c = lax.axis_index("c")
    s = lax.axis_index("s")
    wid = s * NSC + c
    iota = lax.iota(jnp.int32, 16)
    lane8 = iota & 7
    ea_lane = (iota >> 1) & 3
    low_mask = iota < 8

    @pl.when(s < 15)
    def _():
        pltpu.sync_copy(z8_r.at[pl.ds(s * _SLC_A, _SLC_A)],
                        ssum_sp.at[pl.ds(s * _SLC_A, _SLC_A)])

    @pl.when(s == 15)
    def _():
        pltpu.sync_copy(z8_r.at[pl.ds(15 * _SLC_A, _LAST_A + 16)],
                        ssum_sp.at[pl.ds(15 * _SLC_A, _LAST_A + 16)])

    plsc.subcore_barrier()

    def sync_l(j, b):
        e0 = jnp.minimum(wid + NW * j, NCHUNK - 1) * CH
        pltpu.sync_copy(row_r.at[pl.ds(e0, CH)], rowb.at[b])
        pltpu.sync_copy(col_r.at[pl.ds(e0, CH)], colb.at[b])
        pltpu.sync_copy(ea_r.at[pl.ds(e0, CH)], eab.at[b])

    def issue_g(b):
        pltpu.async_copy(s1_r.at[rowb.at[b]], g1.at[b], semG.at[b])
        pltpu.async_copy(s2_r.at[colb.at[b]], g2.at[b], semG.at[b])

    def drain_g(b):
        pltpu.make_async_copy(s1_r.at[pl.ds(0, CH)], g1.at[b],
                              semG.at[b]).wait()
        pltpu.make_async_copy(s2_r.at[pl.ds(0, CH)], g2.at[b],
                              semG.at[b]).wait()

    def edge_loop(bs, bv):
        def edge(e, _):
            ev = jnp.broadcast_to(e, (16,))
            t = g1[bs, e] + g2[bs, e]
            sc = plsc.load_gather(eab, [bv, ev, ea_lane])
            a = t * sc
            a = jnp.maximum(a, NEG * a)
            ex = jnp.exp(a)
            plsc.store_scatter(exb, [ev, lane8], ex, mask=low_mask)
            return 0

        lax.fori_loop(0, CH, edge, 0, unroll=4)

    sync_l(0, 0)
    issue_g(0)

    def chunk_body(j, _):
        b = j & 1
        b1 = 1 - b
        bv = jnp.broadcast_to(b, (16,))
        e0 = (wid + NW * j) * CH
        sync_l(j + 1, b1)
        issue_g(b1)
        drain_g(b)
        edge_loop(b, bv)
        pltpu.sync_copy(exb, ex_r.at[pl.ds(e0, CH)])
        pltpu.sync_copy(exb, ssum_sp.at[rowb.at[b]], add=True)
        return 0

    lax.fori_loop(0, _NJA, chunk_body, 0)
    drain_g(_NJA & 1)

    zv = jnp.broadcast_to(0, (16,))
    cidx = (wid & 3) + _NJA * NW
    e0x = cidx * CH
    pltpu.sync_copy(row_r.at[pl.ds(e0x, CH)], rowb.at[0])
    pltpu.sync_copy(col_r.at[pl.ds(e0x, CH)], colb.at[0])
    pltpu.sync_copy(ea_r.at[pl.ds(e0x, CH)], eab.at[0])
    pltpu.sync_copy(s1_r.at[rowb.at[0]], g1.at[0])
    pltpu.sync_copy(s2_r.at[colb.at[0]], g2.at[0])
    edge_loop(0, zv)

    def redirect(v, _):
        lv = v * 16 + iota
        rv = plsc.load_gather(rowb, [zv, lv])
        rv2 = jnp.where(wid < _NEXTRA_A, rv, N)
        plsc.store_scatter(rowb, [zv, lv], rv2)
        return 0

    lax.fori_loop(0, CH // 16, redirect, 0)
    pltpu.sync_copy(exb, ex_r.at[pl.ds(e0x, CH)])
    pltpu.sync_copy(exb, ssum_sp.at[rowb.at[0]], add=True)

    plsc.subcore_barrier()

    def flush(r0, nrow):
        pltpu.sync_copy(ssum_sp.at[pl.ds(r0, nrow)],
                        ssp_r.at[c, pl.ds(r0, nrow)])

    @pl.when(s < 15)
    def _():
        flush(s * _SLC_A, _SLC_A)

    @pl.when(s == 15)
    def _():
        flush(15 * _SLC_A, _LAST_A)


_sc_a = functools.partial(
    pl.kernel,
    out_type=[
        jax.ShapeDtypeStruct((E, C), jnp.float32),
        jax.ShapeDtypeStruct((2, N, C), jnp.float32),
    ],
    mesh=_mesh,
    compiler_params=_sc_params,
    scratch_types=[
        pltpu.VMEM_SHARED((N + 16, C), jnp.float32),
        pltpu.VMEM((2, CH), jnp.int32),
        pltpu.VMEM((2, CH), jnp.int32),
        pltpu.VMEM((2, CH, 4), jnp.float32),
        pltpu.VMEM((2, CH, 16), jnp.float32),
        pltpu.VMEM((2, CH, 16), jnp.float32),
        pltpu.VMEM((CH, 8), jnp.float32),
        pltpu.SemaphoreType.DMA((2,)),
    ],
)(_sca_body)


_EPI = 125
_NJ = NCHUNK // NT
_NEXTRA = NCHUNK - _NJ * NT


def _scc_body(row_r, col_r, ex_r, ssp_r, xws_r, bias_r, z128_r,
              o_r, ap_r,
              out_sp, rowb, colb, exb, d0b, d1b, gbuf, apb, biasb, semG):
    c = lax.axis_index("c")
    s = lax.axis_index("s")
    iota = lax.iota(jnp.int32, 16)
    lane8 = iota & 7
    lane_e = iota >> 3
    c4 = c * 4
    amask = (lane8 >= c4) & (lane8 < c4 + 4)

    pltpu.sync_copy(z128_r.at[pl.ds(s * 625, 625)],
                    out_sp.at[pl.ds(s * 625, 625)])
    pltpu.sync_copy(bias_r.at[pl.ds(c * 128, 128)], biasb)
    plsc.subcore_barrier()

    def sync_l(j, b):
        e0 = jnp.minimum(s + NT * j, NCHUNK - 1) * CH
        pltpu.sync_copy(row_r.at[pl.ds(e0, CH)], rowb.at[b])
        pltpu.sync_copy(col_r.at[pl.ds(e0, CH)], colb.at[b])
        pltpu.sync_copy(ex_r.at[pl.ds(e0, CH)], exb.at[b])

    def issue_g(b):
        pltpu.async_copy(ssp_r.at[0].at[rowb.at[b]], d0b.at[b], semG.at[b])
        pltpu.async_copy(ssp_r.at[1].at[rowb.at[b]], d1b.at[b], semG.at[b])
        pltpu.async_copy(xws_r.at[c].at[colb.at[b]], gbuf.at[b], semG.at[b])

    def drain_g(b):
        pltpu.make_async_copy(ssp_r.at[0, pl.ds(0, CH)], d0b.at[b],
                              semG.at[b]).wait()
        pltpu.make_async_copy(ssp_r.at[1, pl.ds(0, CH)], d1b.at[b],
                              semG.at[b]).wait()
        pltpu.make_async_copy(xws_r.at[0, pl.ds(0, CH)], gbuf.at[b],
                              semG.at[b]).wait()

    def pair_loop(bv):
        def pair(k, _):
            e = k * 2
            ev = jnp.broadcast_to(e, (16,)) + lane_e
            exv = plsc.load_gather(exb, [bv, ev, lane8])
            d = (plsc.load_gather(d0b, [bv, ev, lane8])
                 + plsc.load_gather(d1b, [bv, ev, lane8]) + 1e-16)
            al = exv / d
            plsc.store_scatter(apb, [bv, ev, lane8 - c4], al, mask=amask)
            return 0

        lax.fori_loop(0, CH // 2, pair, 0, unroll=8)

    def scale_loop(bs, bv):
        def scale(e, _):
            erow = jnp.broadcast_to(e, (16,))
            for h in range(4):
                m = plsc.load_gather(
                    apb, [bv, erow, jnp.broadcast_to(h, (16,))])
                for r in (2 * h, 2 * h + 1):
                    sl = pl.ds(r * 16, 16)
                    gbuf[bs, e, sl] = gbuf[bs, e, sl] * m
            return 0

        lax.fori_loop(0, CH, scale, 0, unroll=4)

    sync_l(0, 0)
    issue_g(0)

    def chunk_body(j, _):
        b = j & 1
        b1 = 1 - b
        bv = jnp.broadcast_to(b, (16,))
        e0 = (s + NT * j) * CH
        sync_l(j + 1, b1)
        issue_g(b1)
        drain_g(b)
        pair_loop(bv)
        scale_loop(b, bv)
        pltpu.sync_copy(apb.at[b], ap_r.at[c, pl.ds(e0, CH)])
        pltpu.sync_copy(gbuf.at[b], out_sp.at[rowb.at[b]], add=True)
        return 0

    lax.fori_loop(0, _NJ, chunk_body, 0)
    drain_g(_NJ & 1)

    zv = jnp.broadcast_to(0, (16,))
    e0x = ((s & 3) + _NJ * NT) * CH
    pltpu.sync_copy(row_r.at[pl.ds(e0x, CH)], rowb.at[0])
    pltpu.sync_copy(col_r.at[pl.ds(e0x, CH)], colb.at[0])
    pltpu.sync_copy(ex_r.at[pl.ds(e0x, CH)], exb.at[0])
    pltpu.sync_copy(ssp_r.at[0].at[rowb.at[0]], d0b.at[0])
    pltpu.sync_copy(ssp_r.at[1].at[rowb.at[0]], d1b.at[0])
    pltpu.sync_copy(xws_r.at[c].at[colb.at[0]], gbuf.at[0])
    pair_loop(zv)
    scale_loop(0, zv)

    def redirect(v, _):
        lv = v * 16 + iota
        rv = plsc.load_gather(rowb, [zv, lv])
        rv2 = jnp.where(s < _NEXTRA, rv, N)
        plsc.store_scatter(rowb, [zv, lv], rv2)
        return 0

    lax.fori_loop(0, CH // 16, redirect, 0)
    pltpu.sync_copy(apb.at[0], ap_r.at[c, pl.ds(e0x, CH)])
    pltpu.sync_copy(gbuf.at[0], out_sp.at[rowb.at[0]], add=True)

    plsc.subcore_barrier()

    def batch(b, _):
        r0 = s * 625 + b * _EPI
        pltpu.sync_copy(out_sp.at[pl.ds(r0, _EPI)],
                        gbuf.at[0].at[pl.ds(0, _EPI)])

        def browe(m, _):
            for r in range(8):
                sl = pl.ds(r * 16, 16)
                gbuf[0, m, sl] = gbuf[0, m, sl] + biasb[pl.ds(r * 16, 16)]
            return 0

        lax.fori_loop(0, _EPI, browe, 0, unroll=2)
        pltpu.sync_copy(gbuf.at[0].at[pl.ds(0, _EPI)],
                        o_r.at[c, pl.ds(r0, _EPI)])
        return 0

    lax.fori_loop(0, 5, batch, 0)


_sc_c = functools.partial(
    pl.kernel,
    out_type=[
        jax.ShapeDtypeStruct((2, N, 128), jnp.float32),
        jax.ShapeDtypeStruct((2, E, 4), jnp.float32),
    ],
    mesh=_mesh,
    compiler_params=_sc_params,
    scratch_types=[
        pltpu.VMEM_SHARED((N + 16, 128), jnp.float32),
        pltpu.VMEM((2, CH), jnp.int32),
        pltpu.VMEM((2, CH), jnp.int32),
        pltpu.VMEM((2, CH, 8), jnp.float32),
        pltpu.VMEM((2, CH, 8), jnp.float32),
        pltpu.VMEM((2, CH, 8), jnp.float32),
        pltpu.VMEM((2, CH, 128), jnp.float32),
        pltpu.VMEM((2, CH, 4), jnp.float32),
        pltpu.VMEM((128,), jnp.float32),
        pltpu.SemaphoreType.DMA((2,)),
    ],
)(_scc_body)


def kernel(x, edge_index, edge_attr, weight, att_weight, bias):
    aw1 = att_weight[0, :, :F]
    aw2 = att_weight[0, :, F:]
    eye = jnp.eye(C, dtype=jnp.float32)
    w1 = (aw1[:, :, None] * eye[:, None, :]).reshape(C * F, C)
    w2 = (aw2[:, :, None] * eye[:, None, :]).reshape(C * F, C)
    zc = jnp.zeros((C * F, C), jnp.float32)
    w12p = jnp.concatenate([w1, zc, w2, zc], axis=1)

    s1p, s2p = _dense_stage_s(x, weight, w12p)
    (xws,) = _dense_stage_x(x, weight)

    row = edge_index[0]
    col = edge_index[1]
    z8 = jnp.zeros((N + 16, C), jnp.float32)
    z128 = jnp.zeros((N, 128), jnp.float32)

    ex, ssp = _sc_a(row, col, edge_attr, s1p, s2p, z8)
    o2, ap2 = _sc_c(row, col, ex, ssp, xws, bias, z128)

    out = jnp.concatenate([o2[0], o2[1]], axis=1)
    alpha = jnp.concatenate([ap2[0], ap2[1]], axis=1)
    return (out, alpha)
